# baseline (speedup 1.0000x reference)
import jax
import jax.numpy as jnp
import numpy as np
from jax import lax
from jax.experimental import pallas as pl
from jax.experimental.pallas import tpu as pltpu
from jax.experimental.pallas import tpu_sc as plsc

VOCAB = 100000
MAX_LEN = 200
DIM = 128
BATCH = 1024

NC = 2
NS = 16
NW = NC * NS
STAGE = 128
STAGES = (BATCH // NW) * MAX_LEN // STAGE
NBUF = 6
LOOKAHEAD = 4
HALF = STAGE // 2


def _build_pe():
    pos = np.arange(MAX_LEN, dtype=np.float32).reshape(-1, 1)
    div = np.power(
        10000.0, np.arange(0, DIM, 2, dtype=np.float32) / DIM)
    ang = (pos / div).astype(np.float32)
    pe = np.zeros((MAX_LEN, DIM), dtype=np.float32)
    pe[:, 0::2] = np.sin(ang)
    pe[:, 1::2] = np.cos(ang)
    return jnp.asarray(pe)


def _sc_body(x_hbm, pe_hbm, table_hbm, out_hbm,
             idx_all, r0, r1, r2, r3, r4, r5, pe_v,
             g0, g1, g2, g3, g4, g5, s0, s1, s2, s3, s4, s5, psem):
    wid = lax.axis_index("s") * NC + lax.axis_index("c")
    rows = (r0, r1, r2, r3, r4, r5)
    gsem = (g0, g1, g2, g3, g4, g5)
    ssem = (s0, s1, s2, s3, s4, s5)
    out_base = wid * (STAGES * STAGE)

    pe_cp = pltpu.async_copy(pe_hbm, pe_v, psem)
    pltpu.sync_copy(x_hbm.at[wid], idx_all)

    BLOCK = 25

    def block_body(k, carry):
        s0_dyn = k * (BLOCK * STAGE)
        pend_g = [None] * NBUF
        pend_s = [[] for _ in range(NBUF)]

        def fire_gather(j):
            b = j % NBUF
            pend_g[b] = [
                pltpu.async_copy(
                    table_hbm.at[idx_all.at[k * BLOCK + j,
                                            pl.ds(h * HALF, HALF)]],
                    rows[b].at[pl.ds(h * HALF, HALF)], gsem[b])
                for h in range(2)]

        for j in range(LOOKAHEAD):
            fire_gather(j)
        for j in range(BLOCK):
            b = j % NBUF
            if j + LOOKAHEAD < BLOCK:
                b2 = (j + LOOKAHEAD) % NBUF
                for cp in pend_s[b2]:
                    cp.wait()
                pend_s[b2] = []
                fire_gather(j + LOOKAHEAD)
            for cp in pend_g[b]:
                cp.wait()

            rows_b = rows[b]
            base = (j * STAGE) % MAX_LEN
            cut = min(STAGE, MAX_LEN - base)

            def run_add(lo, hi):
                segs = []
                if lo < cut:
                    segs.append((lo, min(hi, cut), base))
                if hi > cut:
                    segs.append((max(lo, cut), hi, base - MAX_LEN))
                for (l, h, off) in segs:
                    @plsc.parallel_loop(l, h, unroll=2)
                    def add_body(i, off=off, rows_b=rows_b):
                        for cc in range(DIM // 16):
                            sl = pl.ds(cc * 16, 16)
                            plsc.addupdate(
                                rows_b.at[i, sl], pe_v[i + off, sl])

            for h in range(2):
                run_add(h * HALF, (h + 1) * HALF)
                pend_s[b].append(pltpu.async_copy(
                    rows_b.at[pl.ds(h * HALF, HALF)],
                    out_hbm.at[pl.ds(
                        out_base + s0_dyn + j * STAGE + h * HALF, HALF)],
                    ssem[b]))
        for b in range(NBUF):
            for cp in pend_s[b]:
                cp.wait()
        return carry

    pe_cp.wait()
    lax.fori_loop(0, STAGES // BLOCK, block_body, 0)


@jax.jit
def _run(x2, pe, table):
    mesh = plsc.VectorSubcoreMesh(
        core_axis_name="c", subcore_axis_name="s",
        num_cores=NC, num_subcores=NS)
    f = pl.kernel(
        _sc_body,
        out_type=jax.ShapeDtypeStruct((BATCH * MAX_LEN, DIM), jnp.float32),
        mesh=mesh,
        scratch_types=[
            pltpu.VMEM((STAGES, STAGE), jnp.int32),
            pltpu.VMEM((STAGE, DIM), jnp.float32),
            pltpu.VMEM((STAGE, DIM), jnp.float32),
            pltpu.VMEM((STAGE, DIM), jnp.float32),
            pltpu.VMEM((STAGE, DIM), jnp.float32),
            pltpu.VMEM((STAGE, DIM), jnp.float32),
            pltpu.VMEM((STAGE, DIM), jnp.float32),
            pltpu.VMEM((MAX_LEN, DIM), jnp.float32),
            pltpu.SemaphoreType.DMA,
            pltpu.SemaphoreType.DMA,
            pltpu.SemaphoreType.DMA,
            pltpu.SemaphoreType.DMA,
            pltpu.SemaphoreType.DMA,
            pltpu.SemaphoreType.DMA,
            pltpu.SemaphoreType.DMA,
            pltpu.SemaphoreType.DMA,
            pltpu.SemaphoreType.DMA,
            pltpu.SemaphoreType.DMA,
            pltpu.SemaphoreType.DMA,
            pltpu.SemaphoreType.DMA,
            pltpu.SemaphoreType.DMA,
        ],
    )
    return f(x2, pe, table)


def kernel(x, table):
    x2 = x.reshape(NW, STAGES, STAGE)
    pe = _build_pe()
    out = _run(x2, pe, table)
    return out.reshape(BATCH, MAX_LEN, DIM)


# speedup vs baseline: 1.0030x; 1.0030x over previous
import jax
import jax.numpy as jnp
import numpy as np
from jax import lax
from jax.experimental import pallas as pl
from jax.experimental.pallas import tpu as pltpu
from jax.experimental.pallas import tpu_sc as plsc

VOCAB = 100000
MAX_LEN = 200
DIM = 128
BATCH = 1024

NC = 2
NS = 16
NW = NC * NS
STAGE = 128
STAGES = (BATCH // NW) * MAX_LEN // STAGE
NBUF = 6
LOOKAHEAD = 4
HALF = STAGE // 2


def _build_pe():
    pos = np.arange(MAX_LEN, dtype=np.float32).reshape(-1, 1)
    div = np.power(
        10000.0, np.arange(0, DIM, 2, dtype=np.float32) / DIM)
    ang = (pos / div).astype(np.float32)
    pe = np.zeros((MAX_LEN, DIM), dtype=np.float32)
    pe[:, 0::2] = np.sin(ang)
    pe[:, 1::2] = np.cos(ang)
    return jnp.asarray(pe)


def _sc_body(x_hbm, pe_hbm, table_hbm, out_hbm,
             idx_all, r0, r1, r2, r3, r4, r5, pe_v,
             g0, g1, g2, g3, g4, g5, s0, s1, s2, s3, s4, s5, psem):
    wid = lax.axis_index("s") * NC + lax.axis_index("c")
    rows = (r0, r1, r2, r3, r4, r5)
    gsem = (g0, g1, g2, g3, g4, g5)
    ssem = (s0, s1, s2, s3, s4, s5)
    out_base = wid * (STAGES * STAGE)

    pe_cp = pltpu.async_copy(pe_hbm, pe_v, psem)
    pltpu.sync_copy(x_hbm.at[wid], idx_all)

    BLOCK = 25

    def block_body(k, carry):
        s0_dyn = k * (BLOCK * STAGE)
        pend_g = [None] * NBUF
        pend_s = [[] for _ in range(NBUF)]

        def fire_gather(j):
            b = j % NBUF
            pend_g[b] = [
                pltpu.async_copy(
                    table_hbm.at[idx_all.at[k * BLOCK + j,
                                            pl.ds(h * HALF, HALF)]],
                    rows[b].at[pl.ds(h * HALF, HALF)], gsem[b])
                for h in range(2)]

        for j in range(LOOKAHEAD):
            fire_gather(j)
        for j in range(BLOCK):
            b = j % NBUF
            if j + LOOKAHEAD < BLOCK:
                b2 = (j + LOOKAHEAD) % NBUF
                for cp in pend_s[b2]:
                    cp.wait()
                pend_s[b2] = []
                fire_gather(j + LOOKAHEAD)

            rows_b = rows[b]
            base = (j * STAGE) % MAX_LEN
            cut = min(STAGE, MAX_LEN - base)

            def run_add(lo, hi):
                segs = []
                if lo < cut:
                    segs.append((lo, min(hi, cut), base))
                if hi > cut:
                    segs.append((max(lo, cut), hi, base - MAX_LEN))
                for (l, h, off) in segs:
                    @plsc.parallel_loop(l, h, unroll=2)
                    def add_body(i, off=off, rows_b=rows_b):
                        for cc in range(DIM // 16):
                            sl = pl.ds(cc * 16, 16)
                            plsc.addupdate(
                                rows_b.at[i, sl], pe_v[i + off, sl])

            for h in range(2):
                pend_g[b][h].wait()
                run_add(h * HALF, (h + 1) * HALF)
                pend_s[b].append(pltpu.async_copy(
                    rows_b.at[pl.ds(h * HALF, HALF)],
                    out_hbm.at[pl.ds(
                        out_base + s0_dyn + j * STAGE + h * HALF, HALF)],
                    ssem[b]))
        for b in range(NBUF):
            for cp in pend_s[b]:
                cp.wait()
        return carry

    pe_cp.wait()
    lax.fori_loop(0, STAGES // BLOCK, block_body, 0)


@jax.jit
def _run(x2, pe, table):
    mesh = plsc.VectorSubcoreMesh(
        core_axis_name="c", subcore_axis_name="s",
        num_cores=NC, num_subcores=NS)
    f = pl.kernel(
        _sc_body,
        out_type=jax.ShapeDtypeStruct((BATCH * MAX_LEN, DIM), jnp.float32),
        mesh=mesh,
        scratch_types=[
            pltpu.VMEM((STAGES, STAGE), jnp.int32),
            pltpu.VMEM((STAGE, DIM), jnp.float32),
            pltpu.VMEM((STAGE, DIM), jnp.float32),
            pltpu.VMEM((STAGE, DIM), jnp.float32),
            pltpu.VMEM((STAGE, DIM), jnp.float32),
            pltpu.VMEM((STAGE, DIM), jnp.float32),
            pltpu.VMEM((STAGE, DIM), jnp.float32),
            pltpu.VMEM((MAX_LEN, DIM), jnp.float32),
            pltpu.SemaphoreType.DMA,
            pltpu.SemaphoreType.DMA,
            pltpu.SemaphoreType.DMA,
            pltpu.SemaphoreType.DMA,
            pltpu.SemaphoreType.DMA,
            pltpu.SemaphoreType.DMA,
            pltpu.SemaphoreType.DMA,
            pltpu.SemaphoreType.DMA,
            pltpu.SemaphoreType.DMA,
            pltpu.SemaphoreType.DMA,
            pltpu.SemaphoreType.DMA,
            pltpu.SemaphoreType.DMA,
            pltpu.SemaphoreType.DMA,
        ],
    )
    return f(x2, pe, table)


def kernel(x, table):
    x2 = x.reshape(NW, STAGES, STAGE)
    pe = _build_pe()
    out = _run(x2, pe, table)
    return out.reshape(BATCH, MAX_LEN, DIM)
